# baseline (speedup 1.0000x reference)
import functools

import jax
import jax.numpy as jnp
from jax import lax
from jax.experimental import pallas as pl
from jax.experimental.pallas import tpu as pltpu
from jax.experimental.pallas import tpu_sc as plsc

_VOCAB = 1000000
_EMBED = 64
_BATCH = 16384
_PATH = 20

_NC = 2
_NS = 16
_NW = _NC * _NS
_BPW = _BATCH // _NW
_CHUNK = 128
_NCH = _BPW // _CHUNK
_GROUPS = _BPW // 16


def _sc_dot_body(t_hbm, c_hbm, ww_hbm, wn_hbm, dot_hbm,
                 idx_t, idx_c, rows_a, rows_b, dot_v, sem):
    wid = lax.axis_index("s") * _NC + lax.axis_index("c")
    base = wid * _BPW

    pltpu.sync_copy(t_hbm.at[pl.ds(wid * _NCH, _NCH)], idx_t)
    pltpu.sync_copy(c_hbm.at[pl.ds(wid * _NCH, _NCH)], idx_c)

    copies = []
    for j in range(_NCH):
        copies.append(pltpu.async_copy(
            ww_hbm.at[idx_t.at[j]], rows_a.at[pl.ds(j * _CHUNK, _CHUNK)], sem))
        copies.append(pltpu.async_copy(
            wn_hbm.at[idx_c.at[j]], rows_b.at[pl.ds(j * _CHUNK, _CHUNK)], sem))
    for cp in copies:
        cp.wait()

    lane = lax.iota(jnp.int32, 16)

    def group(g, carry):
        r0 = g * 16
        dsum = jnp.zeros((16,), jnp.float32)
        for r in range(16):
            row = r0 + r
            acc = rows_a[row, pl.ds(0, 16)] * rows_b[row, pl.ds(0, 16)]
            for k in range(1, _EMBED // 16):
                acc = acc + (rows_a[row, pl.ds(k * 16, 16)] *
                             rows_b[row, pl.ds(k * 16, 16)])
            s = jnp.sum(acc)
            dsum = jnp.where(lane == r, s, dsum)
        dot_v[pl.ds(r0, 16)] = dsum
        return carry

    lax.fori_loop(0, _GROUPS, group, 0)
    pltpu.sync_copy(dot_v, dot_hbm.at[pl.ds(base, _BPW)])


_sc_dot = functools.partial(
    pl.kernel,
    out_type=jax.ShapeDtypeStruct((_BATCH,), jnp.float32),
    mesh=plsc.VectorSubcoreMesh(core_axis_name="c", subcore_axis_name="s"),
    compiler_params=pltpu.CompilerParams(
        needs_layout_passes=False, use_tc_tiling_on_sc=False),
    scratch_types=[
        pltpu.VMEM((_NCH, _CHUNK), jnp.int32),
        pltpu.VMEM((_NCH, _CHUNK), jnp.int32),
        pltpu.VMEM((_BPW, _EMBED), jnp.float32),
        pltpu.VMEM((_BPW, _EMBED), jnp.float32),
        pltpu.VMEM((_BPW,), jnp.float32),
        pltpu.SemaphoreType.DMA,
    ],
)(_sc_dot_body)


def _tc_loss_body(dot_ref, codes_ref, out_ref):
    d = dot_ref[...]
    c = jnp.sum(codes_ref[...].astype(jnp.float32), axis=-1)
    sp_pos = jnp.maximum(d, 0.0) + jnp.log1p(jnp.exp(-jnp.abs(d)))
    sp_neg = sp_pos - d
    out_ref[...] = (float(_PATH) - c) * sp_neg + c * sp_pos


def kernel(target, context, codes, W_word, W_node):
    t2 = target.astype(jnp.int32).reshape(_NW * _NCH, _CHUNK)
    c2 = context.astype(jnp.int32).reshape(_NW * _NCH, _CHUNK)
    dot = _sc_dot(t2, c2, W_word, W_node)
    loss2 = pl.pallas_call(
        _tc_loss_body,
        out_shape=jax.ShapeDtypeStruct((128, 128), jnp.float32),
    )(dot.reshape(128, 128), codes.reshape(128, 128, _PATH))
    return loss2.reshape(_BATCH)
